# baseline (speedup 1.0000x reference)
import functools

import jax
import jax.numpy as jnp
from jax import lax
from jax.experimental import pallas as pl
from jax.experimental.pallas import tpu as pltpu
from jax.experimental.pallas import tpu_sc as plsc

NC = 2
NS = 16
LANES = 16
C = 224
NBUF = 3


def _make_kernel(N, D, K):
    total_chunks = (K + C - 1) // C
    n0 = int(round(total_chunks * 0.571 / NS)) * NS
    n0 = min(max(n0, NS), (total_chunks // NS) * NS)
    n1 = ((total_chunks - n0 + NS - 1) // NS) * NS
    SHARE = (n0 // NS * C, n1 // NS * C)
    START = (0, NS * SHARE[0])
    KP = NS * (SHARE[0] + SHARE[1])
    NCH = (SHARE[0] // C, SHARE[1] // C)
    VALID = K - (START[1] + (NS - 1) * SHARE[1])
    TAIL = VALID - (NCH[1] - 1) * C
    n_vec = D // LANES
    assert 0 < TAIL <= C and VALID % 8 == 0 and (SHARE[1] - VALID) % LANES == 0

    mesh = plsc.VectorSubcoreMesh(
        core_axis_name="c", subcore_axis_name="s",
        num_cores=NC, num_subcores=NS)

    @functools.partial(
        pl.kernel,
        out_type=jax.ShapeDtypeStruct((K, D), jnp.float32),
        mesh=mesh,
        scratch_types=[
            pltpu.VMEM((max(SHARE),), jnp.int32),
            pltpu.VMEM((max(SHARE),), jnp.float32),
            pltpu.VMEM((NBUF, C, D), jnp.float32),
            pltpu.SemaphoreType.DMA,
            pltpu.SemaphoreType.DMA,
            pltpu.SemaphoreType.DMA,
            pltpu.SemaphoreType.DMA,
            pltpu.SemaphoreType.DMA,
            pltpu.SemaphoreType.DMA,
        ],
    )
    def gather_scale(feat_hbm, idx_hbm, scores_hbm, out_hbm,
                     idx_v, sc_v, rows_v, g0, g1, g2, o0, o1, o2):
        cid = lax.axis_index("c")
        sid = lax.axis_index("s")
        gsem = (g0, g1, g2)
        osem = (o0, o1, o2)

        def emit(c):
            share, nch = SHARE[c], NCH[c]
            base = START[c] + sid * share
            ragged = (c == 1)
            last_w = (sid == NS - 1)

            if not ragged:
                pltpu.sync_copy(idx_hbm.at[pl.ds(base, share)],
                                idx_v.at[pl.ds(0, share)])
                pltpu.sync_copy(scores_hbm.at[pl.ds(base, share)],
                                sc_v.at[pl.ds(0, share)])
            else:
                @pl.when(~last_w)
                def _():
                    pltpu.sync_copy(idx_hbm.at[pl.ds(base, share)],
                                    idx_v.at[pl.ds(0, share)])
                    pltpu.sync_copy(scores_hbm.at[pl.ds(base, share)],
                                    sc_v.at[pl.ds(0, share)])

                @pl.when(last_w)
                def _():
                    pltpu.sync_copy(idx_hbm.at[pl.ds(base, VALID)],
                                    idx_v.at[pl.ds(0, VALID)])
                    pltpu.sync_copy(scores_hbm.at[pl.ds(base, VALID)],
                                    sc_v.at[pl.ds(0, VALID)])
                    for t in range((share - VALID) // LANES):
                        idx_v[pl.ds(VALID + t * LANES, LANES)] = \
                            jnp.zeros((LANES,), jnp.int32)

            def start_gather(k):
                b = k % NBUF
                return pltpu.async_copy(
                    feat_hbm.at[idx_v.at[pl.ds(k * C, C)]],
                    rows_v.at[b], gsem[b])

            def scale_chunk(k):
                b = k % NBUF

                def body(g, _):
                    sv = sc_v[pl.ds(k * C + g * LANES, LANES)]
                    for i in range(LANES):
                        s = sv[i]
                        for j in range(n_vec):
                            sl = pl.ds(j * LANES, LANES)
                            rows_v[b, g * LANES + i, sl] = \
                                rows_v[b, g * LANES + i, sl] * s
                    return _

                lax.fori_loop(0, C // LANES, body, None)

            gdesc = {k: start_gather(k) for k in range(min(NBUF - 1, nch))}
            odesc = {}
            for k in range(nch):
                b = k % NBUF
                off = base + k * C
                if k + NBUF - 1 < nch:
                    if k - 1 in odesc:
                        odesc.pop(k - 1).wait()
                    gdesc[k + NBUF - 1] = start_gather(k + NBUF - 1)
                gdesc.pop(k).wait()
                scale_chunk(k)
                if k < nch - 1 or not ragged:
                    odesc[k] = pltpu.async_copy(
                        rows_v.at[b], out_hbm.at[pl.ds(off, C)], osem[b])
                else:
                    @pl.when(~last_w)
                    def _():
                        pltpu.async_copy(
                            rows_v.at[b], out_hbm.at[pl.ds(off, C)],
                            osem[b]).wait()

                    @pl.when(last_w)
                    def _():
                        pltpu.async_copy(
                            rows_v.at[b].at[pl.ds(0, TAIL)],
                            out_hbm.at[pl.ds(off, TAIL)], osem[b]).wait()
            for k in sorted(odesc):
                odesc.pop(k).wait()

        @pl.when(cid == 0)
        def _():
            emit(0)

        @pl.when(cid == 1)
        def _():
            emit(1)

    return gather_scale, KP


def kernel(feat, select_idx, scores):
    N, D = feat.shape
    K = select_idx.shape[0]
    fn, _ = _make_kernel(N, D, K)
    return fn(feat, select_idx.astype(jnp.int32), scores)


# speedup vs baseline: 1.0558x; 1.0558x over previous
import functools

import jax
import jax.numpy as jnp
from jax import lax
from jax.experimental import pallas as pl
from jax.experimental.pallas import tpu as pltpu
from jax.experimental.pallas import tpu_sc as plsc

NC = 2
NS = 16
NW = NC * NS
LANES = 16
NBUF = 3


def _make_kernel(N, D, K):
    align = 8 * NW
    KP = ((K + align - 1) // align) * align
    BPW = KP // NW
    C = 224
    while BPW % C != 0:
        C -= 16
    NCHUNK = BPW // C
    TAIL = K - ((NW - 1) * BPW + (NCHUNK - 1) * C)
    VALID = K - (NW - 1) * BPW
    n_vec = D // LANES

    mesh = plsc.VectorSubcoreMesh(
        core_axis_name="c", subcore_axis_name="s",
        num_cores=NC, num_subcores=NS)

    @functools.partial(
        pl.kernel,
        out_type=jax.ShapeDtypeStruct((K, D), jnp.float32),
        mesh=mesh,
        scratch_types=[
            pltpu.VMEM((BPW,), jnp.int32),
            pltpu.VMEM((BPW,), jnp.float32),
            pltpu.VMEM((NBUF, C, D), jnp.float32),
            pltpu.VMEM((LANES,), jnp.int32),
            pltpu.SemaphoreType.DMA,
            pltpu.SemaphoreType.DMA,
            pltpu.SemaphoreType.DMA,
            pltpu.SemaphoreType.DMA,
            pltpu.SemaphoreType.DMA,
            pltpu.SemaphoreType.DMA,
        ],
    )
    def gather_scale(feat_hbm, idx_hbm, scores_hbm, ng_hbm, out_hbm,
                     idx_v, sc_v, rows_v, ng_v, g0, g1, g2, o0, o1, o2):
        cid = lax.axis_index("c")
        sid = lax.axis_index("s")
        wid = sid * NC + cid
        base = wid * BPW
        gsem = (g0, g1, g2)
        osem = (o0, o1, o2)
        last_w = wid == NW - 1

        if VALID == BPW:
            pltpu.sync_copy(idx_hbm.at[pl.ds(base, BPW)], idx_v)
            pltpu.sync_copy(scores_hbm.at[pl.ds(base, BPW)], sc_v)
        else:
            @pl.when(~last_w)
            def _():
                pltpu.sync_copy(idx_hbm.at[pl.ds(base, BPW)], idx_v)
                pltpu.sync_copy(scores_hbm.at[pl.ds(base, BPW)], sc_v)

            @pl.when(last_w)
            def _():
                pltpu.sync_copy(idx_hbm.at[pl.ds(base, VALID)],
                                idx_v.at[pl.ds(0, VALID)])
                pltpu.sync_copy(scores_hbm.at[pl.ds(base, VALID)],
                                sc_v.at[pl.ds(0, VALID)])
                for t in range((BPW - VALID) // LANES):
                    idx_v[pl.ds(VALID + t * LANES, LANES)] = \
                        jnp.zeros((LANES,), jnp.int32)

        def start_gather(k):
            b = k % NBUF
            return pltpu.async_copy(
                feat_hbm.at[idx_v.at[pl.ds(k * C, C)]],
                rows_v.at[b], gsem[b])

        pltpu.sync_copy(ng_hbm, ng_v)
        n_groups = ng_v[...][0]

        def scale_chunk(k):
            b = k % NBUF

            @pl.loop(0, n_groups)
            def _(g):
                sv = sc_v[pl.ds(k * C + g * LANES, LANES)]
                for i in range(LANES):
                    s = sv[i]
                    for j in range(n_vec):
                        sl = pl.ds(j * LANES, LANES)
                        rows_v[b, g * LANES + i, sl] = \
                            rows_v[b, g * LANES + i, sl] * s

        gdesc = {k: start_gather(k) for k in range(min(NBUF - 1, NCHUNK))}
        odesc = {}
        for k in range(NCHUNK):
            b = k % NBUF
            off = base + k * C
            if k + NBUF - 1 < NCHUNK:
                if k - 1 in odesc:
                    odesc.pop(k - 1).wait()
                gdesc[k + NBUF - 1] = start_gather(k + NBUF - 1)
            gdesc.pop(k).wait()
            scale_chunk(k)
            if k < NCHUNK - 1 or TAIL == C:
                odesc[k] = pltpu.async_copy(
                    rows_v.at[b], out_hbm.at[pl.ds(off, C)], osem[b])
            else:
                @pl.when(~last_w)
                def _():
                    pltpu.async_copy(
                        rows_v.at[b], out_hbm.at[pl.ds(off, C)],
                        osem[b]).wait()

                @pl.when(last_w)
                def _():
                    pltpu.async_copy(
                        rows_v.at[b].at[pl.ds(0, TAIL)],
                        out_hbm.at[pl.ds(off, TAIL)], osem[b]).wait()
        for k in sorted(odesc):
            odesc.pop(k).wait()

    return gather_scale, C


def kernel(feat, select_idx, scores):
    N, D = feat.shape
    K = select_idx.shape[0]
    fn, C = _make_kernel(N, D, K)
    ng = jnp.full((LANES,), C // LANES, jnp.int32)
    return fn(feat, select_idx.astype(jnp.int32), scores, ng)
